# Initial kernel scaffold; baseline (speedup 1.0000x reference)
import functools

import jax
import jax.numpy as jnp
from jax import lax
from jax.experimental import pallas as pl
from jax.experimental.pallas import tpu as pltpu
from jax.experimental.pallas import tpu_sc as plsc

F32 = jnp.float32
I32 = jnp.int32

_NC = 2
_NS = 16
_B = 128


def _build_sc_segsum(N, R, E_pad, acc_rows, with_deg):
    NW = _NC * _NS
    ept = E_pad // NW
    nb = ept // _B
    rt = acc_rows // _NS
    nz = rt // _B

    mesh = plsc.VectorSubcoreMesh(core_axis_name="c", subcore_axis_name="s")
    out_type = [jax.ShapeDtypeStruct((_NC, acc_rows, 64), F32)]
    scratch = [
        pltpu.VMEM((_B,), I32),
        pltpu.VMEM((_B,), I32),
        pltpu.VMEM((_B,), I32),
        pltpu.VMEM((_B,), I32),
        pltpu.VMEM((_B,), I32),
        pltpu.VMEM((_B, 64), F32),
        pltpu.VMEM((_B, 16), F32),
        pltpu.VMEM_SHARED((acc_rows, 64), F32),
        pltpu.SemaphoreType.DMA,
    ]
    if with_deg:
        out_type.append(jax.ShapeDtypeStruct((_NC, acc_rows, 16), F32))
        scratch.append(pltpu.VMEM_SHARED((acc_rows, 16), F32))

    def body(xr, srcr, dstr, typr, *rest):
        if with_deg:
            (outB, outD, src_v, dst_v, typ_v, idx_v, pair_v, rows_v, ones_v,
             acc_s, sem, deg_s) = rest
        else:
            (outB, src_v, dst_v, typ_v, idx_v, pair_v, rows_v, ones_v,
             acc_s, sem) = rest
        cid = lax.axis_index("c")
        sid = lax.axis_index("s")
        wid = sid * _NC + cid

        zv = jnp.zeros((16,), F32)

        def zero_body(i, carry):
            for k in range(4):
                rows_v[i, pl.ds(16 * k, 16)] = zv
            ones_v[i] = zv
            return carry

        lax.fori_loop(0, _B, zero_body, 0)

        rbase = sid * rt
        for k in range(nz):
            pltpu.sync_copy(rows_v, acc_s.at[pl.ds(rbase + _B * k, _B)])
            if with_deg:
                pltpu.sync_copy(ones_v, deg_s.at[pl.ds(rbase + _B * k, _B)])
        plsc.subcore_barrier()

        if with_deg:
            ov = jnp.ones((16,), F32)

            def ones_body(i, carry):
                ones_v[i] = ov
                return carry

            lax.fori_loop(0, _B, ones_body, 0)

        ebase = wid * ept

        def batch(j, carry):
            off = ebase + j * _B
            pltpu.sync_copy(srcr.at[pl.ds(off, _B)], src_v)
            pltpu.sync_copy(dstr.at[pl.ds(off, _B)], dst_v)
            pltpu.sync_copy(typr.at[pl.ds(off, _B)], typ_v)
            for k in range(_B // 16):
                sl = pl.ds(16 * k, 16)
                idx_v[sl] = src_v[sl] * 2 + cid
                pair_v[sl] = typ_v[sl] * N + dst_v[sl]
            pltpu.async_copy(xr.at[idx_v], rows_v, sem).wait()
            pltpu.sync_copy(rows_v, acc_s.at[pair_v], add=True)
            if with_deg:
                @pl.when((j % 2) == cid)
                def _():
                    pltpu.sync_copy(ones_v, deg_s.at[pair_v], add=True)
            return carry

        lax.fori_loop(0, nb, batch, 0)
        plsc.subcore_barrier()

        pltpu.sync_copy(acc_s.at[pl.ds(rbase, rt)],
                        outB.at[cid, pl.ds(rbase, rt)])
        if with_deg:
            pltpu.sync_copy(deg_s.at[pl.ds(rbase, rt)],
                            outD.at[cid, pl.ds(rbase, rt)])

    return pl.kernel(body, out_type=out_type, mesh=mesh,
                     scratch_types=scratch)


def _build_tc_layer(N, R, acc_rows, BN, last_layer):
    nblk = N // BN

    def body(b00, b01, b10, b11, degb, xb, wt, wb, wr, bb, ob):
        n0 = 1.0 / jnp.maximum(degb[:, 0:1], 1.0)
        n1 = 1.0 / jnp.maximum(degb[:, 1:2], 1.0)
        t0 = (jnp.dot(b00[0], wt[0], preferred_element_type=F32)
              + jnp.dot(b10[0], wb[0], preferred_element_type=F32))
        t1 = (jnp.dot(b01[0], wt[1], preferred_element_type=F32)
              + jnp.dot(b11[0], wb[1], preferred_element_type=F32))
        agg = t0 * n0 + t1 * n1
        agg = agg + jnp.dot(xb[...], wr[...], preferred_element_type=F32)
        agg = agg + bb[0:1, :]
        if last_layer:
            s = jnp.sum(agg * agg, axis=1, keepdims=True)
            ob[...] = agg / jnp.maximum(jnp.sqrt(s), 1e-12)
        else:
            ob[...] = jnp.maximum(agg, 0.0)

    in_specs = [
        pl.BlockSpec((1, BN, 64), lambda i: (0, i, 0)),
        pl.BlockSpec((1, BN, 64), lambda i: (0, i + nblk, 0)),
        pl.BlockSpec((1, BN, 64), lambda i: (1, i, 0)),
        pl.BlockSpec((1, BN, 64), lambda i: (1, i + nblk, 0)),
        pl.BlockSpec((BN, 128), lambda i: (i, 0)),
        pl.BlockSpec((BN, 128), lambda i: (i, 0)),
        pl.BlockSpec((R, 64, 128), lambda i: (0, 0, 0)),
        pl.BlockSpec((R, 64, 128), lambda i: (0, 0, 0)),
        pl.BlockSpec((128, 128), lambda i: (0, 0)),
        pl.BlockSpec((8, 128), lambda i: (0, 0)),
    ]
    return pl.pallas_call(
        body,
        grid=(nblk,),
        in_specs=in_specs,
        out_specs=pl.BlockSpec((BN, 128), lambda i: (i, 0)),
        out_shape=jax.ShapeDtypeStruct((N, 128), F32),
    )


def kernel(x, edge_index, edge_type, W1, Wroot1, b1, W2, Wroot2, b2):
    N, D = x.shape
    E = edge_index.shape[1]
    R = W1.shape[0]
    assert D == 128 and R == 2

    NW = _NC * _NS
    ept = -(-E // (NW * _B)) * _B
    E_pad = ept * NW
    acc_rows = -(-(R * N + 1) // (_NS * _B)) * (_NS * _B)
    pad = E_pad - E

    src = edge_index[0]
    dst = edge_index[1]
    if pad:
        src = jnp.concatenate([src, jnp.zeros((pad,), I32)])
        dst = jnp.concatenate([dst, jnp.full((pad,), N, I32)])
        edge_type = jnp.concatenate([edge_type, jnp.full((pad,), R - 1, I32)])

    xr = x.reshape(2 * N, 64)
    sc_deg = _build_sc_segsum(N, R, E_pad, acc_rows, True)
    outB1, outD = sc_deg(xr, src, dst, edge_type)

    degfull = outD[0, :R * N, 0] + outD[1, :R * N, 0]
    degPad = (jnp.zeros((N, 128), F32)
              .at[:, 0].set(degfull[:N])
              .at[:, 1].set(degfull[N:]))

    BN = 400
    tc = _build_tc_layer(N, R, acc_rows, BN, False)
    h = tc(outB1, outB1, outB1, outB1, degPad, x,
           W1[:, :64, :], W1[:, 64:, :], Wroot1,
           jnp.zeros((8, 128), F32).at[0].set(b1))

    hr = h.reshape(2 * N, 64)
    sc2 = _build_sc_segsum(N, R, E_pad, acc_rows, False)
    (outB2,) = sc2(hr, src, dst, edge_type)

    tc2 = _build_tc_layer(N, R, acc_rows, BN, True)
    out = tc2(outB2, outB2, outB2, outB2, degPad, h,
              W2[:, :64, :], W2[:, 64:, :], Wroot2,
              jnp.zeros((8, 128), F32).at[0].set(b2))
    return out


# speedup vs baseline: 12.9889x; 12.9889x over previous
import functools

import jax
import jax.numpy as jnp
from jax import lax
from jax.experimental import pallas as pl
from jax.experimental.pallas import tpu as pltpu
from jax.experimental.pallas import tpu_sc as plsc

F32 = jnp.float32
I32 = jnp.int32

_NC = 2
_NS = 16
_B = 128


def _build_sc_segsum(N, R, E_pad, acc_rows, with_deg):
    ept = E_pad // _NS
    nb = ept // _B
    rt = acc_rows // _NS
    nz = rt // _B

    mesh = plsc.VectorSubcoreMesh(core_axis_name="c", subcore_axis_name="s")
    out_type = [jax.ShapeDtypeStruct((_NC, acc_rows, 64), F32)]
    scratch = [
        pltpu.VMEM((_B,), I32),
        pltpu.VMEM((_B,), I32),
        pltpu.VMEM((_B,), I32),
        pltpu.VMEM((_B,), I32),
        pltpu.VMEM((_B,), I32),
        pltpu.VMEM((_B, 64), F32),
        pltpu.VMEM((_B, 16), F32),
        pltpu.VMEM_SHARED((acc_rows, 64), F32),
        pltpu.SemaphoreType.DMA,
    ]
    if with_deg:
        out_type.append(jax.ShapeDtypeStruct((_NC, acc_rows, 16), F32))
        scratch.append(pltpu.VMEM_SHARED((acc_rows, 16), F32))

    def body(xr, srcr, dstr, typr, *rest):
        if with_deg:
            (outB, outD, src_v, dst_v, typ_v, idx_v, pair_v, rows_v, ones_v,
             acc_s, sem, deg_s) = rest
        else:
            (outB, src_v, dst_v, typ_v, idx_v, pair_v, rows_v, ones_v,
             acc_s, sem) = rest
        cid = lax.axis_index("c")
        sid = lax.axis_index("s")

        zv = jnp.zeros((16,), F32)

        def zero_body(i, carry):
            for k in range(4):
                rows_v[i, pl.ds(16 * k, 16)] = zv
            ones_v[i] = zv
            return carry

        lax.fori_loop(0, _B, zero_body, 0)

        rbase = sid * rt
        for k in range(nz):
            pltpu.sync_copy(rows_v, acc_s.at[pl.ds(rbase + _B * k, _B)])
            if with_deg:
                pltpu.sync_copy(ones_v, deg_s.at[pl.ds(rbase + _B * k, _B)])
        plsc.subcore_barrier()

        if with_deg:
            ov = jnp.ones((16,), F32)

            def ones_body(i, carry):
                ones_v[i] = ov
                return carry

            lax.fori_loop(0, _B, ones_body, 0)

        ebase = sid * ept

        def batch(j, carry):
            off = ebase + j * _B
            pltpu.sync_copy(srcr.at[pl.ds(off, _B)], src_v)
            pltpu.sync_copy(dstr.at[pl.ds(off, _B)], dst_v)
            pltpu.sync_copy(typr.at[pl.ds(off, _B)], typ_v)
            for k in range(_B // 16):
                sl = pl.ds(16 * k, 16)
                idx_v[sl] = src_v[sl] * 2 + cid
                pair_v[sl] = typ_v[sl] * N + dst_v[sl]
            pltpu.async_copy(xr.at[idx_v], rows_v, sem).wait()
            pltpu.sync_copy(rows_v, acc_s.at[pair_v], add=True)
            if with_deg:
                @pl.when((j % 2) == cid)
                def _():
                    pltpu.sync_copy(ones_v, deg_s.at[pair_v], add=True)
            return carry

        lax.fori_loop(0, nb, batch, 0)
        plsc.subcore_barrier()

        pltpu.sync_copy(acc_s.at[pl.ds(rbase, rt)],
                        outB.at[cid, pl.ds(rbase, rt)])
        if with_deg:
            pltpu.sync_copy(deg_s.at[pl.ds(rbase, rt)],
                            outD.at[cid, pl.ds(rbase, rt)])

    return pl.kernel(body, out_type=out_type, mesh=mesh,
                     scratch_types=scratch,
                     compiler_params=pltpu.CompilerParams(
                         use_tc_tiling_on_sc=False))


def _build_tc_layer(N, R, acc_rows, BN, last_layer):
    nblk = N // BN

    def body(b00, b01, b10, b11, degb, xb, wt, wb, wr, bb, ob):
        n0 = 1.0 / jnp.maximum(degb[:, 0:1], 1.0)
        n1 = 1.0 / jnp.maximum(degb[:, 1:2], 1.0)
        t0 = (jnp.dot(b00[0], wt[0], preferred_element_type=F32)
              + jnp.dot(b10[0], wb[0], preferred_element_type=F32))
        t1 = (jnp.dot(b01[0], wt[1], preferred_element_type=F32)
              + jnp.dot(b11[0], wb[1], preferred_element_type=F32))
        agg = t0 * n0 + t1 * n1
        agg = agg + jnp.dot(xb[...], wr[...], preferred_element_type=F32)
        agg = agg + bb[0:1, :]
        if last_layer:
            s = jnp.sum(agg * agg, axis=1, keepdims=True)
            ob[...] = agg / jnp.maximum(jnp.sqrt(s), 1e-12)
        else:
            ob[...] = jnp.maximum(agg, 0.0)

    in_specs = [
        pl.BlockSpec((1, BN, 64), lambda i: (0, i, 0)),
        pl.BlockSpec((1, BN, 64), lambda i: (0, i + nblk, 0)),
        pl.BlockSpec((1, BN, 64), lambda i: (1, i, 0)),
        pl.BlockSpec((1, BN, 64), lambda i: (1, i + nblk, 0)),
        pl.BlockSpec((BN, 128), lambda i: (i, 0)),
        pl.BlockSpec((BN, 128), lambda i: (i, 0)),
        pl.BlockSpec((R, 64, 128), lambda i: (0, 0, 0)),
        pl.BlockSpec((R, 64, 128), lambda i: (0, 0, 0)),
        pl.BlockSpec((128, 128), lambda i: (0, 0)),
        pl.BlockSpec((8, 128), lambda i: (0, 0)),
    ]
    return pl.pallas_call(
        body,
        grid=(nblk,),
        in_specs=in_specs,
        out_specs=pl.BlockSpec((BN, 128), lambda i: (i, 0)),
        out_shape=jax.ShapeDtypeStruct((N, 128), F32),
    )


def kernel(x, edge_index, edge_type, W1, Wroot1, b1, W2, Wroot2, b2):
    N, D = x.shape
    E = edge_index.shape[1]
    R = W1.shape[0]
    assert D == 128 and R == 2

    E_pad = -(-E // (_NS * _B)) * (_NS * _B)
    acc_rows = -(-(R * N + 1) // (_NS * _B)) * (_NS * _B)
    pad = E_pad - E

    src = edge_index[0]
    dst = edge_index[1]
    if pad:
        src = jnp.concatenate([src, jnp.zeros((pad,), I32)])
        dst = jnp.concatenate([dst, jnp.full((pad,), N, I32)])
        edge_type = jnp.concatenate([edge_type, jnp.full((pad,), R - 1, I32)])

    xr = x.reshape(2 * N, 64)
    sc_deg = _build_sc_segsum(N, R, E_pad, acc_rows, True)
    outB1, outD = sc_deg(xr, src, dst, edge_type)

    degfull = outD[0, :R * N, 0] + outD[1, :R * N, 0]
    degPad = (jnp.zeros((N, 128), F32)
              .at[:, 0].set(degfull[:N])
              .at[:, 1].set(degfull[N:]))

    BN = 400
    tc = _build_tc_layer(N, R, acc_rows, BN, False)
    h = tc(outB1, outB1, outB1, outB1, degPad, x,
           W1[:, :64, :], W1[:, 64:, :], Wroot1,
           jnp.zeros((8, 128), F32).at[0].set(b1))

    hr = h.reshape(2 * N, 64)
    sc2 = _build_sc_segsum(N, R, E_pad, acc_rows, False)
    (outB2,) = sc2(hr, src, dst, edge_type)

    tc2 = _build_tc_layer(N, R, acc_rows, BN, True)
    out = tc2(outB2, outB2, outB2, outB2, degPad, h,
              W2[:, :64, :], W2[:, 64:, :], Wroot2,
              jnp.zeros((8, 128), F32).at[0].set(b2))
    return out


# speedup vs baseline: 13.6690x; 1.0524x over previous
import functools

import jax
import jax.numpy as jnp
from jax import lax
from jax.experimental import pallas as pl
from jax.experimental.pallas import tpu as pltpu
from jax.experimental.pallas import tpu_sc as plsc

F32 = jnp.float32
I32 = jnp.int32

_NC = 2
_NS = 16
_B = 128


_SC_EDGES = 1024
_SQ = _SC_EDGES // _B


def _build_sc_segsum(N, R, E_pad, acc_rows, with_deg):
    ept = E_pad // _NS
    nb = ept // _B
    nsc = nb // _SQ
    rt = acc_rows // _NS
    nz = rt // _B
    assert nb % 2 == 0 and nsc * _SQ == nb

    mesh = plsc.VectorSubcoreMesh(core_axis_name="c", subcore_axis_name="s")
    out_type = [jax.ShapeDtypeStruct((_NC, acc_rows, 64), F32)]
    scratch = [
        pltpu.VMEM((2, _SQ, _B), I32),
        pltpu.VMEM((2, _SQ, _B), I32),
        [pltpu.VMEM((_B, 64), F32)] * 2,
        pltpu.VMEM((_B, 16), F32),
        pltpu.VMEM_SHARED((acc_rows, 64), F32),
        [pltpu.SemaphoreType.DMA] * 2,
        [pltpu.SemaphoreType.DMA] * 2,
        pltpu.SemaphoreType.DMA,
    ]
    if with_deg:
        out_type.append(jax.ShapeDtypeStruct((_NC, acc_rows, 16), F32))
        scratch.append(pltpu.VMEM_SHARED((acc_rows, 16), F32))

    def body(xr, idxr, pairr, *rest):
        if with_deg:
            (outB, outD, idx_c, pair_c, rows_b, ones_v,
             acc_s, sem_g, sem_s, sem_d, deg_s) = rest
        else:
            (outB, idx_c, pair_c, rows_b, ones_v,
             acc_s, sem_g, sem_s, sem_d) = rest
        cid = lax.axis_index("c")
        sid = lax.axis_index("s")

        zv = jnp.zeros((16,), F32)

        def zero_body(i, carry):
            for k in range(4):
                rows_b[0][i, pl.ds(16 * k, 16)] = zv
            ones_v[i] = zv
            return carry

        lax.fori_loop(0, _B, zero_body, 0)

        rbase = sid * rt
        for k in range(nz):
            pltpu.sync_copy(rows_b[0], acc_s.at[pl.ds(rbase + _B * k, _B)])
            if with_deg:
                pltpu.sync_copy(ones_v, deg_s.at[pl.ds(rbase + _B * k, _B)])
        plsc.subcore_barrier()

        if with_deg:
            ov = jnp.ones((16,), F32)

            def ones_body(i, carry):
                ones_v[i] = ov
                return carry

            lax.fori_loop(0, _B, ones_body, 0)

        def load_idx(s):
            bs = lax.rem(s, 2)
            gsc = sid * nsc + s
            pltpu.sync_copy(idxr.at[cid, gsc], idx_c.at[bs])
            pltpu.sync_copy(pairr.at[gsc], pair_c.at[bs])

        def fire_gather(j, b):
            s = lax.div(j, _SQ)
            q = lax.rem(j, _SQ)
            pltpu.async_copy(xr.at[idx_c.at[lax.rem(s, 2), q]],
                             rows_b[b], sem_g[b])

        def drain_gather(b):
            pltpu.make_async_copy(xr.at[idx_c.at[0, 0]], rows_b[b],
                                  sem_g[b]).wait()

        def drain_deg():
            pltpu.make_async_copy(ones_v, deg_s.at[pair_c.at[0, 0]],
                                  sem_d).wait()

        def fire_scatter(j, b):
            s = lax.div(j, _SQ)
            q = lax.rem(j, _SQ)
            pidx = pair_c.at[lax.rem(s, 2), q]
            pltpu.async_copy(rows_b[b], acc_s.at[pidx], sem_s[b], add=True)
            if with_deg:
                @pl.when(lax.rem(j, 2) == cid)
                def _():
                    @pl.when(j >= 2)
                    def _():
                        drain_deg()
                    pltpu.async_copy(ones_v, deg_s.at[pidx], sem_d, add=True)

        def drain_scatter(b):
            pltpu.make_async_copy(rows_b[b], acc_s.at[pair_c.at[0, 0]],
                                  sem_s[b]).wait()

        load_idx(0)
        fire_gather(0, 0)

        def pipeline(i, carry):
            for b in range(2):
                j = 2 * i + b
                jn = j + 1

                @pl.when(jnp.logical_and(lax.rem(jn, _SQ) == 0, jn < nb))
                def _():
                    load_idx(lax.div(jn, _SQ))
                drain_gather(b)
                fire_scatter(j, b)

                @pl.when(j > 0)
                def _():
                    drain_scatter(1 - b)

                @pl.when(jn < nb)
                def _():
                    fire_gather(jn, 1 - b)
            return carry

        lax.fori_loop(0, nb // 2, pipeline, 0)
        drain_scatter(1)
        if with_deg:
            drain_deg()
        plsc.subcore_barrier()

        pltpu.sync_copy(acc_s.at[pl.ds(rbase, rt)],
                        outB.at[cid, pl.ds(rbase, rt)])
        if with_deg:
            pltpu.sync_copy(deg_s.at[pl.ds(rbase, rt)],
                            outD.at[cid, pl.ds(rbase, rt)])

    return pl.kernel(body, out_type=out_type, mesh=mesh,
                     scratch_types=scratch,
                     compiler_params=pltpu.CompilerParams(
                         use_tc_tiling_on_sc=False))


def _build_tc_layer(N, R, acc_rows, BN, last_layer):
    nblk = N // BN

    def body(b00, b01, b10, b11, degb, xb, wt, wb, wr, bb, ob):
        n0 = 1.0 / jnp.maximum(degb[:, 0:1], 1.0)
        n1 = 1.0 / jnp.maximum(degb[:, 1:2], 1.0)
        t0 = (jnp.dot(b00[0], wt[0], preferred_element_type=F32)
              + jnp.dot(b10[0], wb[0], preferred_element_type=F32))
        t1 = (jnp.dot(b01[0], wt[1], preferred_element_type=F32)
              + jnp.dot(b11[0], wb[1], preferred_element_type=F32))
        agg = t0 * n0 + t1 * n1
        agg = agg + jnp.dot(xb[...], wr[...], preferred_element_type=F32)
        agg = agg + bb[0:1, :]
        if last_layer:
            s = jnp.sum(agg * agg, axis=1, keepdims=True)
            ob[...] = agg / jnp.maximum(jnp.sqrt(s), 1e-12)
        else:
            ob[...] = jnp.maximum(agg, 0.0)

    in_specs = [
        pl.BlockSpec((1, BN, 64), lambda i: (0, i, 0)),
        pl.BlockSpec((1, BN, 64), lambda i: (0, i + nblk, 0)),
        pl.BlockSpec((1, BN, 64), lambda i: (1, i, 0)),
        pl.BlockSpec((1, BN, 64), lambda i: (1, i + nblk, 0)),
        pl.BlockSpec((BN, 128), lambda i: (i, 0)),
        pl.BlockSpec((BN, 128), lambda i: (i, 0)),
        pl.BlockSpec((R, 64, 128), lambda i: (0, 0, 0)),
        pl.BlockSpec((R, 64, 128), lambda i: (0, 0, 0)),
        pl.BlockSpec((128, 128), lambda i: (0, 0)),
        pl.BlockSpec((8, 128), lambda i: (0, 0)),
    ]
    return pl.pallas_call(
        body,
        grid=(nblk,),
        in_specs=in_specs,
        out_specs=pl.BlockSpec((BN, 128), lambda i: (i, 0)),
        out_shape=jax.ShapeDtypeStruct((N, 128), F32),
    )


def kernel(x, edge_index, edge_type, W1, Wroot1, b1, W2, Wroot2, b2):
    N, D = x.shape
    E = edge_index.shape[1]
    R = W1.shape[0]
    assert D == 128 and R == 2

    E_pad = -(-E // (_NS * _SC_EDGES)) * (_NS * _SC_EDGES)
    acc_rows = -(-(R * N + 1) // (_NS * _B)) * (_NS * _B)
    pad = E_pad - E

    src = edge_index[0]
    dst = edge_index[1]
    if pad:
        src = jnp.concatenate([src, jnp.zeros((pad,), I32)])
        dst = jnp.concatenate([dst, jnp.full((pad,), N, I32)])
        edge_type = jnp.concatenate([edge_type, jnp.full((pad,), R - 1, I32)])

    nsc_all = E_pad // _SC_EDGES
    idxR = jnp.stack([2 * src, 2 * src + 1]).reshape(2, nsc_all, _SQ, _B)
    pairR = (edge_type * N + dst).reshape(nsc_all, _SQ, _B)

    xr = x.reshape(2 * N, 64)
    sc_deg = _build_sc_segsum(N, R, E_pad, acc_rows, True)
    outB1, outD = sc_deg(xr, idxR, pairR)

    degfull = outD[0, :R * N, 0] + outD[1, :R * N, 0]
    degPad = (jnp.zeros((N, 128), F32)
              .at[:, 0].set(degfull[:N])
              .at[:, 1].set(degfull[N:]))

    BN = 400
    tc = _build_tc_layer(N, R, acc_rows, BN, False)
    h = tc(outB1, outB1, outB1, outB1, degPad, x,
           W1[:, :64, :], W1[:, 64:, :], Wroot1,
           jnp.zeros((8, 128), F32).at[0].set(b1))

    hr = h.reshape(2 * N, 64)
    sc2 = _build_sc_segsum(N, R, E_pad, acc_rows, False)
    (outB2,) = sc2(hr, idxR, pairR)

    tc2 = _build_tc_layer(N, R, acc_rows, BN, True)
    out = tc2(outB2, outB2, outB2, outB2, degPad, h,
              W2[:, :64, :], W2[:, 64:, :], Wroot2,
              jnp.zeros((8, 128), F32).at[0].set(b2))
    return out


# speedup vs baseline: 13.9764x; 1.0225x over previous
import functools

import jax
import jax.numpy as jnp
from jax import lax
from jax.experimental import pallas as pl
from jax.experimental.pallas import tpu as pltpu
from jax.experimental.pallas import tpu_sc as plsc

F32 = jnp.float32
I32 = jnp.int32

_NC = 2
_NS = 16
_B = 128


_SC_EDGES = 1024
_SQ = _SC_EDGES // _B


def _build_sc_segsum(N, R, E_pad, acc_rows, with_deg, sg=1, nbuf=2):
    ept = E_pad // _NS
    nb = ept // (_B * sg)
    nsc = nb * sg // _SQ
    rt = acc_rows // _NS
    nz = rt // _B
    assert nb % nbuf == 0 and nsc * _SQ == nb * sg and _SQ % sg == 0
    assert 2 <= nbuf <= 8

    mesh = plsc.VectorSubcoreMesh(core_axis_name="c", subcore_axis_name="s")
    out_type = [jax.ShapeDtypeStruct((_NC, acc_rows, 64), F32)]
    scratch = [
        pltpu.VMEM((2, _SQ, _B), I32),
        pltpu.VMEM((2, _SQ, _B), I32),
        [pltpu.VMEM((sg, _B, 64), F32)] * nbuf,
        pltpu.VMEM((_B, 16), F32),
        pltpu.VMEM_SHARED((acc_rows, 64), F32),
        [pltpu.SemaphoreType.DMA] * nbuf,
        [pltpu.SemaphoreType.DMA] * nbuf,
        pltpu.SemaphoreType.DMA,
    ]
    if with_deg:
        out_type.append(jax.ShapeDtypeStruct((_NC, acc_rows, 16), F32))
        scratch.append(pltpu.VMEM_SHARED((acc_rows, 16), F32))

    def body(xr, idxr, pairr, *rest):
        if with_deg:
            (outB, outD, idx_c, pair_c, rows_b, ones_v,
             acc_s, sem_g, sem_s, sem_d, deg_s) = rest
        else:
            (outB, idx_c, pair_c, rows_b, ones_v,
             acc_s, sem_g, sem_s, sem_d) = rest
        cid = lax.axis_index("c")
        sid = lax.axis_index("s")

        zv = jnp.zeros((16,), F32)

        def zero_body(i, carry):
            for k in range(4):
                rows_b[0][0, i, pl.ds(16 * k, 16)] = zv
            ones_v[i] = zv
            return carry

        lax.fori_loop(0, _B, zero_body, 0)

        rbase = sid * rt
        z128 = rows_b[0].at[0]
        for k in range(nz):
            pltpu.sync_copy(z128, acc_s.at[pl.ds(rbase + _B * k, _B)])
            if with_deg:
                pltpu.sync_copy(ones_v, deg_s.at[pl.ds(rbase + _B * k, _B)])
        plsc.subcore_barrier()

        if with_deg:
            ov = jnp.ones((16,), F32)

            def ones_body(i, carry):
                ones_v[i] = ov
                return carry

            lax.fori_loop(0, _B, ones_body, 0)

        def load_idx(s):
            bs = lax.rem(s, 2)
            gsc = sid * nsc + s
            pltpu.sync_copy(idxr.at[cid, gsc], idx_c.at[bs])
            pltpu.sync_copy(pairr.at[gsc], pair_c.at[bs])

        def _ix(ref, bs, q):
            return ref.at[bs, q] if sg == 1 else ref.at[bs, pl.ds(q, sg)]

        def _rows(b):
            return rows_b[b].at[0] if sg == 1 else rows_b[b]

        def fire_gather(j, b):
            jq = j * sg
            s = lax.div(jq, _SQ)
            q = lax.rem(jq, _SQ)
            pltpu.async_copy(xr.at[_ix(idx_c, lax.rem(s, 2), q)],
                             _rows(b), sem_g[b])

        def drain_gather(b):
            pltpu.make_async_copy(xr.at[_ix(idx_c, 0, 0)],
                                  _rows(b), sem_g[b]).wait()

        def drain_deg():
            pltpu.make_async_copy(ones_v, deg_s.at[pair_c.at[0, 0]],
                                  sem_d).wait()

        def fire_scatter(j, b):
            jq = j * sg
            s = lax.div(jq, _SQ)
            q = lax.rem(jq, _SQ)
            bs = lax.rem(s, 2)
            pidx = _ix(pair_c, bs, q)
            pltpu.async_copy(_rows(b), acc_s.at[pidx], sem_s[b], add=True)
            if with_deg:
                @pl.when(lax.rem(j, 2) == cid)
                def _():
                    for k in range(sg):
                        @pl.when(j >= 2)
                        def _():
                            drain_deg()
                        pltpu.async_copy(ones_v, deg_s.at[pair_c.at[bs, q + k]],
                                         sem_d, add=True)

        def drain_scatter(b):
            pltpu.make_async_copy(_rows(b),
                                  acc_s.at[_ix(pair_c, 0, 0)],
                                  sem_s[b]).wait()

        load_idx(0)
        for w in range(nbuf - 1):
            fire_gather(w, w)

        def pipeline(i, carry):
            for w in range(nbuf):
                j = i * nbuf + w
                wprev = (w - 1) % nbuf
                jf = j + nbuf - 1

                drain_gather(w)
                fire_scatter(j, w)

                @pl.when(j > 0)
                def _():
                    drain_scatter(wprev)

                @pl.when(jf < nb)
                def _():
                    @pl.when(lax.rem(jf * sg, _SQ) == 0)
                    def _():
                        load_idx(lax.div(jf * sg, _SQ))
                    fire_gather(jf, wprev)
            return carry

        lax.fori_loop(0, nb // nbuf, pipeline, 0)
        drain_scatter((nb - 1) % nbuf)
        if with_deg:
            for k in range(sg):
                drain_deg()
        plsc.subcore_barrier()

        pltpu.sync_copy(acc_s.at[pl.ds(rbase, rt)],
                        outB.at[cid, pl.ds(rbase, rt)])
        if with_deg:
            pltpu.sync_copy(deg_s.at[pl.ds(rbase, rt)],
                            outD.at[cid, pl.ds(rbase, rt)])

    return pl.kernel(body, out_type=out_type, mesh=mesh,
                     scratch_types=scratch,
                     compiler_params=pltpu.CompilerParams(
                         use_tc_tiling_on_sc=False))


def _build_tc_layer(N, R, acc_rows, BN, last_layer):
    nblk = N // BN

    def body(b00, b01, b10, b11, degb, xb, wt, wb, wr, bb, ob):
        n0 = 1.0 / jnp.maximum(degb[:, 0:1], 1.0)
        n1 = 1.0 / jnp.maximum(degb[:, 1:2], 1.0)
        t0 = (jnp.dot(b00[0], wt[0], preferred_element_type=F32)
              + jnp.dot(b10[0], wb[0], preferred_element_type=F32))
        t1 = (jnp.dot(b01[0], wt[1], preferred_element_type=F32)
              + jnp.dot(b11[0], wb[1], preferred_element_type=F32))
        agg = t0 * n0 + t1 * n1
        agg = agg + jnp.dot(xb[...], wr[...], preferred_element_type=F32)
        agg = agg + bb[0:1, :]
        if last_layer:
            s = jnp.sum(agg * agg, axis=1, keepdims=True)
            ob[...] = agg / jnp.maximum(jnp.sqrt(s), 1e-12)
        else:
            ob[...] = jnp.maximum(agg, 0.0)

    in_specs = [
        pl.BlockSpec((1, BN, 64), lambda i: (0, i, 0)),
        pl.BlockSpec((1, BN, 64), lambda i: (0, i + nblk, 0)),
        pl.BlockSpec((1, BN, 64), lambda i: (1, i, 0)),
        pl.BlockSpec((1, BN, 64), lambda i: (1, i + nblk, 0)),
        pl.BlockSpec((BN, 128), lambda i: (i, 0)),
        pl.BlockSpec((BN, 128), lambda i: (i, 0)),
        pl.BlockSpec((R, 64, 128), lambda i: (0, 0, 0)),
        pl.BlockSpec((R, 64, 128), lambda i: (0, 0, 0)),
        pl.BlockSpec((128, 128), lambda i: (0, 0)),
        pl.BlockSpec((8, 128), lambda i: (0, 0)),
    ]
    return pl.pallas_call(
        body,
        grid=(nblk,),
        in_specs=in_specs,
        out_specs=pl.BlockSpec((BN, 128), lambda i: (i, 0)),
        out_shape=jax.ShapeDtypeStruct((N, 128), F32),
    )


def kernel(x, edge_index, edge_type, W1, Wroot1, b1, W2, Wroot2, b2):
    N, D = x.shape
    E = edge_index.shape[1]
    R = W1.shape[0]
    assert D == 128 and R == 2

    E_pad = -(-E // (_NS * _SC_EDGES)) * (_NS * _SC_EDGES)
    acc_rows = -(-(R * N + 1) // (_NS * _B)) * (_NS * _B)
    pad = E_pad - E

    src = edge_index[0]
    dst = edge_index[1]
    if pad:
        src = jnp.concatenate([src, jnp.zeros((pad,), I32)])
        dst = jnp.concatenate([dst, jnp.full((pad,), N, I32)])
        edge_type = jnp.concatenate([edge_type, jnp.full((pad,), R - 1, I32)])

    nsc_all = E_pad // _SC_EDGES
    idxR = jnp.stack([2 * src, 2 * src + 1]).reshape(2, nsc_all, _SQ, _B)
    pairR = (edge_type * N + dst).reshape(nsc_all, _SQ, _B)

    xr = x.reshape(2 * N, 64)
    sc_deg = _build_sc_segsum(N, R, E_pad, acc_rows, True)
    outB1, outD = sc_deg(xr, idxR, pairR)

    degfull = outD[0, :R * N, 0] + outD[1, :R * N, 0]
    degPad = (jnp.zeros((N, 128), F32)
              .at[:, 0].set(degfull[:N])
              .at[:, 1].set(degfull[N:]))

    BN = 400
    tc = _build_tc_layer(N, R, acc_rows, BN, False)
    h = tc(outB1, outB1, outB1, outB1, degPad, x,
           W1[:, :64, :], W1[:, 64:, :], Wroot1,
           jnp.zeros((8, 128), F32).at[0].set(b1))

    hr = h.reshape(2 * N, 64)
    sc2 = _build_sc_segsum(N, R, E_pad, acc_rows, False, nbuf=4)
    (outB2,) = sc2(hr, idxR, pairR)

    tc2 = _build_tc_layer(N, R, acc_rows, BN, True)
    out = tc2(outB2, outB2, outB2, outB2, degPad, h,
              W2[:, :64, :], W2[:, 64:, :], Wroot2,
              jnp.zeros((8, 128), F32).at[0].set(b2))
    return out


# speedup vs baseline: 14.2948x; 1.0228x over previous
import functools

import jax
import jax.numpy as jnp
from jax import lax
from jax.experimental import pallas as pl
from jax.experimental.pallas import tpu as pltpu
from jax.experimental.pallas import tpu_sc as plsc

F32 = jnp.float32
I32 = jnp.int32

_NC = 2
_NS = 16
_B = 128


_SC_EDGES = 1024
_SQ = _SC_EDGES // _B


def _build_sc_segsum(N, R, E_pad, acc_rows, with_deg, sg=1, nbuf=2):
    ept = E_pad // _NS
    nb = ept // (_B * sg)
    nsc = nb * sg // _SQ
    rt = acc_rows // _NS
    nz = rt // _B
    assert nb % nbuf == 0 and nsc * _SQ == nb * sg and _SQ % sg == 0
    assert 2 <= nbuf <= 8

    mesh = plsc.VectorSubcoreMesh(core_axis_name="c", subcore_axis_name="s")
    out_type = [jax.ShapeDtypeStruct((_NC, acc_rows, 64), F32)]
    scratch = [
        pltpu.VMEM((2, 2, _SQ, _B), I32),
        [pltpu.VMEM((sg, _B, 64), F32)] * nbuf,
        pltpu.VMEM((_B, 16), F32),
        pltpu.VMEM_SHARED((acc_rows, 64), F32),
        [pltpu.SemaphoreType.DMA] * nbuf,
        [pltpu.SemaphoreType.DMA] * nbuf,
        pltpu.SemaphoreType.DMA,
    ]
    if with_deg:
        out_type.append(jax.ShapeDtypeStruct((_NC, acc_rows, 16), F32))
        scratch.append(pltpu.VMEM_SHARED((acc_rows, 16), F32))

    def body(xr, combr, *rest):
        if with_deg:
            (outB, outD, comb_c, rows_b, ones_v,
             acc_s, sem_g, sem_s, sem_d, deg_s) = rest
        else:
            (outB, comb_c, rows_b, ones_v,
             acc_s, sem_g, sem_s, sem_d) = rest
        cid = lax.axis_index("c")
        sid = lax.axis_index("s")

        zv = jnp.zeros((16,), F32)

        def zero_body(i, carry):
            for k in range(4):
                rows_b[0][0, i, pl.ds(16 * k, 16)] = zv
            ones_v[i] = zv
            return carry

        lax.fori_loop(0, _B, zero_body, 0)

        rbase = sid * rt
        z128 = rows_b[0].at[0]
        for k in range(nz):
            pltpu.sync_copy(z128, acc_s.at[pl.ds(rbase + _B * k, _B)])
            if with_deg:
                pltpu.sync_copy(ones_v, deg_s.at[pl.ds(rbase + _B * k, _B)])
        plsc.subcore_barrier()

        if with_deg:
            ov = jnp.ones((16,), F32)

            def ones_body(i, carry):
                ones_v[i] = ov
                return carry

            lax.fori_loop(0, _B, ones_body, 0)

        def load_idx(s):
            bs = lax.rem(s, 2)
            gsc = sid * nsc + s
            pltpu.sync_copy(combr.at[cid, gsc], comb_c.at[bs])

        def _ix(which, bs, q):
            if sg == 1:
                return comb_c.at[bs, which, q]
            return comb_c.at[bs, which, pl.ds(q, sg)]

        def _rows(b):
            return rows_b[b].at[0] if sg == 1 else rows_b[b]

        def fire_gather(j, b):
            jq = j * sg
            s = lax.div(jq, _SQ)
            q = lax.rem(jq, _SQ)
            pltpu.async_copy(xr.at[_ix(0, lax.rem(s, 2), q)],
                             _rows(b), sem_g[b])

        def drain_gather(b):
            pltpu.make_async_copy(xr.at[_ix(0, 0, 0)],
                                  _rows(b), sem_g[b]).wait()

        def drain_deg():
            pltpu.make_async_copy(ones_v, deg_s.at[comb_c.at[0, 1, 0]],
                                  sem_d).wait()

        def fire_scatter(j, b):
            jq = j * sg
            s = lax.div(jq, _SQ)
            q = lax.rem(jq, _SQ)
            bs = lax.rem(s, 2)
            pidx = _ix(1, bs, q)
            pltpu.async_copy(_rows(b), acc_s.at[pidx], sem_s[b], add=True)
            if with_deg:
                @pl.when(lax.rem(j, 2) == cid)
                def _():
                    for k in range(sg):
                        @pl.when(j >= 2)
                        def _():
                            drain_deg()
                        pltpu.async_copy(ones_v,
                                         deg_s.at[comb_c.at[bs, 1, q + k]],
                                         sem_d, add=True)

        def drain_scatter(b):
            pltpu.make_async_copy(_rows(b),
                                  acc_s.at[_ix(1, 0, 0)],
                                  sem_s[b]).wait()

        load_idx(0)
        for w in range(nbuf - 1):
            fire_gather(w, w)

        def pipeline(i, carry):
            for w in range(nbuf):
                j = i * nbuf + w
                wprev = (w - 1) % nbuf
                jf = j + nbuf - 1

                drain_gather(w)
                fire_scatter(j, w)

                @pl.when(j > 0)
                def _():
                    drain_scatter(wprev)

                @pl.when(jf < nb)
                def _():
                    @pl.when(lax.rem(jf * sg, _SQ) == 0)
                    def _():
                        load_idx(lax.div(jf * sg, _SQ))
                    fire_gather(jf, wprev)
            return carry

        lax.fori_loop(0, nb // nbuf, pipeline, 0)
        drain_scatter((nb - 1) % nbuf)
        if with_deg:
            for k in range(sg):
                drain_deg()
        plsc.subcore_barrier()

        pltpu.sync_copy(acc_s.at[pl.ds(rbase, rt)],
                        outB.at[cid, pl.ds(rbase, rt)])
        if with_deg:
            pltpu.sync_copy(deg_s.at[pl.ds(rbase, rt)],
                            outD.at[cid, pl.ds(rbase, rt)])

    return pl.kernel(body, out_type=out_type, mesh=mesh,
                     scratch_types=scratch,
                     compiler_params=pltpu.CompilerParams(
                         use_tc_tiling_on_sc=False))


def _build_tc_layer(N, R, acc_rows, BN, last_layer):
    nblk = N // BN

    def body(b00, b01, b10, b11, degb, xb, wt, wb, wr, bb, ob):
        n0 = 1.0 / jnp.maximum(degb[:, 0:1], 1.0)
        n1 = 1.0 / jnp.maximum(degb[:, 1:2], 1.0)
        t0 = (jnp.dot(b00[0], wt[0], preferred_element_type=F32)
              + jnp.dot(b10[0], wb[0], preferred_element_type=F32))
        t1 = (jnp.dot(b01[0], wt[1], preferred_element_type=F32)
              + jnp.dot(b11[0], wb[1], preferred_element_type=F32))
        agg = t0 * n0 + t1 * n1
        agg = agg + jnp.dot(xb[...], wr[...], preferred_element_type=F32)
        agg = agg + bb[0:1, :]
        if last_layer:
            s = jnp.sum(agg * agg, axis=1, keepdims=True)
            ob[...] = agg / jnp.maximum(jnp.sqrt(s), 1e-12)
        else:
            ob[...] = jnp.maximum(agg, 0.0)

    in_specs = [
        pl.BlockSpec((1, BN, 64), lambda i: (0, i, 0)),
        pl.BlockSpec((1, BN, 64), lambda i: (0, i + nblk, 0)),
        pl.BlockSpec((1, BN, 64), lambda i: (1, i, 0)),
        pl.BlockSpec((1, BN, 64), lambda i: (1, i + nblk, 0)),
        pl.BlockSpec((BN, 128), lambda i: (i, 0)),
        pl.BlockSpec((BN, 128), lambda i: (i, 0)),
        pl.BlockSpec((R, 64, 128), lambda i: (0, 0, 0)),
        pl.BlockSpec((R, 64, 128), lambda i: (0, 0, 0)),
        pl.BlockSpec((128, 128), lambda i: (0, 0)),
        pl.BlockSpec((8, 128), lambda i: (0, 0)),
    ]
    return pl.pallas_call(
        body,
        grid=(nblk,),
        in_specs=in_specs,
        out_specs=pl.BlockSpec((BN, 128), lambda i: (i, 0)),
        out_shape=jax.ShapeDtypeStruct((N, 128), F32),
    )


def kernel(x, edge_index, edge_type, W1, Wroot1, b1, W2, Wroot2, b2):
    N, D = x.shape
    E = edge_index.shape[1]
    R = W1.shape[0]
    assert D == 128 and R == 2

    E_pad = -(-E // (_NS * _SC_EDGES)) * (_NS * _SC_EDGES)
    acc_rows = -(-(R * N + 1) // (_NS * _B)) * (_NS * _B)
    pad = E_pad - E

    src = edge_index[0]
    dst = edge_index[1]
    if pad:
        src = jnp.concatenate([src, jnp.zeros((pad,), I32)])
        dst = jnp.concatenate([dst, jnp.full((pad,), N, I32)])
        edge_type = jnp.concatenate([edge_type, jnp.full((pad,), R - 1, I32)])

    nsc_all = E_pad // _SC_EDGES
    pairs = (edge_type * N + dst).reshape(nsc_all, _SQ, _B)
    combR = jnp.stack([
        jnp.stack([(2 * src + c).reshape(nsc_all, _SQ, _B), pairs], axis=1)
        for c in range(2)])

    xr = x.reshape(2 * N, 64)
    sc_deg = _build_sc_segsum(N, R, E_pad, acc_rows, True)
    outB1, outD = sc_deg(xr, combR)

    degfull = outD[0, :R * N, 0] + outD[1, :R * N, 0]
    degPad = (jnp.zeros((N, 128), F32)
              .at[:, 0].set(degfull[:N])
              .at[:, 1].set(degfull[N:]))

    BN = 400
    tc = _build_tc_layer(N, R, acc_rows, BN, False)
    h = tc(outB1, outB1, outB1, outB1, degPad, x,
           W1[:, :64, :], W1[:, 64:, :], Wroot1,
           jnp.zeros((8, 128), F32).at[0].set(b1))

    hr = h.reshape(2 * N, 64)
    sc2 = _build_sc_segsum(N, R, E_pad, acc_rows, False, nbuf=4)
    (outB2,) = sc2(hr, combR)

    tc2 = _build_tc_layer(N, R, acc_rows, BN, True)
    out = tc2(outB2, outB2, outB2, outB2, degPad, h,
              W2[:, :64, :], W2[:, 64:, :], Wroot2,
              jnp.zeros((8, 128), F32).at[0].set(b2))
    return out


# speedup vs baseline: 14.3864x; 1.0064x over previous
import functools

import jax
import jax.numpy as jnp
from jax import lax
from jax.experimental import pallas as pl
from jax.experimental.pallas import tpu as pltpu
from jax.experimental.pallas import tpu_sc as plsc

F32 = jnp.float32
I32 = jnp.int32

_NC = 2
_NS = 16
_B = 128


_SC_EDGES = 1024
_SQ = _SC_EDGES // _B


def _build_sc_segsum(N, R, E_pad, acc_rows, with_deg, sg=1, nbuf=2):
    ept = E_pad // _NS
    nb = ept // (_B * sg)
    nsc = nb * sg // _SQ
    rt = acc_rows // _NS
    nz = rt // _B
    assert nb % nbuf == 0 and nsc * _SQ == nb * sg and _SQ % sg == 0
    assert 2 <= nbuf <= 8

    mesh = plsc.VectorSubcoreMesh(core_axis_name="c", subcore_axis_name="s")
    out_type = [jax.ShapeDtypeStruct((_NC, acc_rows, 64), F32)]
    scratch = [
        pltpu.VMEM((2, 2, _SQ, _B), I32),
        [pltpu.VMEM((sg, _B, 64), F32)] * nbuf,
        pltpu.VMEM((_B, 16), F32),
        pltpu.VMEM_SHARED((acc_rows, 64), F32),
        [pltpu.SemaphoreType.DMA] * nbuf,
        [pltpu.SemaphoreType.DMA] * nbuf,
        pltpu.SemaphoreType.DMA,
    ]
    if with_deg:
        out_type.append(jax.ShapeDtypeStruct((_NC, acc_rows, 16), F32))
        scratch.append(pltpu.VMEM_SHARED((acc_rows, 16), F32))

    def body(xr, combr, *rest):
        if with_deg:
            (outB, outD, comb_c, rows_b, ones_v,
             acc_s, sem_g, sem_s, sem_d, deg_s) = rest
        else:
            (outB, comb_c, rows_b, ones_v,
             acc_s, sem_g, sem_s, sem_d) = rest
        cid = lax.axis_index("c")
        sid = lax.axis_index("s")

        zv = jnp.zeros((16,), F32)

        def zero_body(i, carry):
            for k in range(4):
                rows_b[0][0, i, pl.ds(16 * k, 16)] = zv
            ones_v[i] = zv
            return carry

        lax.fori_loop(0, _B, zero_body, 0)

        rbase = sid * rt
        z128 = rows_b[0].at[0]
        for k in range(nz):
            pltpu.sync_copy(z128, acc_s.at[pl.ds(rbase + _B * k, _B)])
            if with_deg:
                pltpu.sync_copy(ones_v, deg_s.at[pl.ds(rbase + _B * k, _B)])
        plsc.subcore_barrier()

        if with_deg:
            ov = jnp.ones((16,), F32)

            def ones_body(i, carry):
                ones_v[i] = ov
                return carry

            lax.fori_loop(0, _B, ones_body, 0)

        def load_idx(s):
            bs = lax.rem(s, 2)
            gsc = sid * nsc + s
            pltpu.sync_copy(combr.at[cid, gsc], comb_c.at[bs])

        def _ix(which, bs, q):
            if sg == 1:
                return comb_c.at[bs, which, q]
            return comb_c.at[bs, which, pl.ds(q, sg)]

        def _rows(b):
            return rows_b[b].at[0] if sg == 1 else rows_b[b]

        def fire_gather(j, b):
            jq = j * sg
            s = lax.div(jq, _SQ)
            q = lax.rem(jq, _SQ)
            pltpu.async_copy(xr.at[_ix(0, lax.rem(s, 2), q)],
                             _rows(b), sem_g[b])

        def drain_gather(b):
            pltpu.make_async_copy(xr.at[_ix(0, 0, 0)],
                                  _rows(b), sem_g[b]).wait()

        def drain_deg():
            pltpu.make_async_copy(ones_v, deg_s.at[comb_c.at[0, 1, 0]],
                                  sem_d).wait()

        def fire_scatter(j, b):
            jq = j * sg
            s = lax.div(jq, _SQ)
            q = lax.rem(jq, _SQ)
            bs = lax.rem(s, 2)
            pidx = _ix(1, bs, q)
            pltpu.async_copy(_rows(b), acc_s.at[pidx], sem_s[b], add=True)
            if with_deg:
                @pl.when(lax.rem(j, 2) == cid)
                def _():
                    for k in range(sg):
                        @pl.when(j >= 2)
                        def _():
                            drain_deg()
                        pltpu.async_copy(ones_v,
                                         deg_s.at[comb_c.at[bs, 1, q + k]],
                                         sem_d, add=True)

        def drain_scatter(b):
            pltpu.make_async_copy(_rows(b),
                                  acc_s.at[_ix(1, 0, 0)],
                                  sem_s[b]).wait()

        load_idx(0)
        for w in range(nbuf - 1):
            fire_gather(w, w)

        def pipeline(i, carry):
            for w in range(nbuf):
                j = i * nbuf + w
                wprev = (w - 1) % nbuf
                jf = j + nbuf - 1

                drain_gather(w)
                fire_scatter(j, w)

                @pl.when(j > 0)
                def _():
                    drain_scatter(wprev)

                @pl.when(jf < nb)
                def _():
                    @pl.when(lax.rem(jf * sg, _SQ) == 0)
                    def _():
                        load_idx(lax.div(jf * sg, _SQ))
                    fire_gather(jf, wprev)
            return carry

        lax.fori_loop(0, nb // nbuf, pipeline, 0)
        drain_scatter((nb - 1) % nbuf)
        if with_deg:
            for k in range(sg):
                drain_deg()
        plsc.subcore_barrier()

        pltpu.sync_copy(acc_s.at[pl.ds(rbase, rt)],
                        outB.at[cid, pl.ds(rbase, rt)])
        if with_deg:
            pltpu.sync_copy(deg_s.at[pl.ds(rbase, rt)],
                            outD.at[cid, pl.ds(rbase, rt)])

    return pl.kernel(body, out_type=out_type, mesh=mesh,
                     scratch_types=scratch,
                     compiler_params=pltpu.CompilerParams(
                         use_tc_tiling_on_sc=False))


def _build_tc_layer(N, R, acc_rows, BN, last_layer):
    nblk = N // BN

    def body(b00, b01, b10, b11, degb, xb, wt, wb, wr, bb, ob):
        n0 = 1.0 / jnp.maximum(degb[:, 0:1], 1.0)
        n1 = 1.0 / jnp.maximum(degb[:, 1:2], 1.0)
        t0 = (jnp.dot(b00[0], wt[0], preferred_element_type=F32)
              + jnp.dot(b10[0], wb[0], preferred_element_type=F32))
        t1 = (jnp.dot(b01[0], wt[1], preferred_element_type=F32)
              + jnp.dot(b11[0], wb[1], preferred_element_type=F32))
        agg = t0 * n0 + t1 * n1
        agg = agg + jnp.dot(xb[...], wr[...], preferred_element_type=F32)
        agg = agg + bb[0:1, :]
        if last_layer:
            s = jnp.sum(agg * agg, axis=1, keepdims=True)
            ob[...] = agg / jnp.maximum(jnp.sqrt(s), 1e-12)
        else:
            ob[...] = jnp.maximum(agg, 0.0)

    in_specs = [
        pl.BlockSpec((1, BN, 64), lambda i: (0, i, 0)),
        pl.BlockSpec((1, BN, 64), lambda i: (0, i + nblk, 0)),
        pl.BlockSpec((1, BN, 64), lambda i: (1, i, 0)),
        pl.BlockSpec((1, BN, 64), lambda i: (1, i + nblk, 0)),
        pl.BlockSpec((BN, 128), lambda i: (i, 0)),
        pl.BlockSpec((BN, 128), lambda i: (i, 0)),
        pl.BlockSpec((R, 64, 128), lambda i: (0, 0, 0)),
        pl.BlockSpec((R, 64, 128), lambda i: (0, 0, 0)),
        pl.BlockSpec((128, 128), lambda i: (0, 0)),
        pl.BlockSpec((8, 128), lambda i: (0, 0)),
    ]
    return pl.pallas_call(
        body,
        grid=(nblk,),
        in_specs=in_specs,
        out_specs=pl.BlockSpec((BN, 128), lambda i: (i, 0)),
        out_shape=jax.ShapeDtypeStruct((N, 128), F32),
    )


def kernel(x, edge_index, edge_type, W1, Wroot1, b1, W2, Wroot2, b2):
    N, D = x.shape
    E = edge_index.shape[1]
    R = W1.shape[0]
    assert D == 128 and R == 2

    E_pad = -(-E // (_NS * _SC_EDGES)) * (_NS * _SC_EDGES)
    acc_rows = -(-(R * N + 1) // (_NS * _B)) * (_NS * _B)
    pad = E_pad - E

    src = edge_index[0]
    dst = edge_index[1]
    if pad:
        src = jnp.concatenate([src, jnp.zeros((pad,), I32)])
        dst = jnp.concatenate([dst, jnp.full((pad,), N, I32)])
        edge_type = jnp.concatenate([edge_type, jnp.full((pad,), R - 1, I32)])

    nsc_all = E_pad // _SC_EDGES
    pairs = (edge_type * N + dst).reshape(nsc_all, _SQ, _B)
    combR = jnp.stack([
        jnp.stack([(2 * src + c).reshape(nsc_all, _SQ, _B), pairs], axis=1)
        for c in range(2)])

    xr = x.reshape(2 * N, 64)
    sc_deg = _build_sc_segsum(N, R, E_pad, acc_rows, True)
    outB1, outD = sc_deg(xr, combR)

    degfull = outD[0, :R * N, 0] + outD[1, :R * N, 0]
    degPad = (jnp.zeros((N, 128), F32)
              .at[:, 0].set(degfull[:N])
              .at[:, 1].set(degfull[N:]))

    BN = 400
    tc = _build_tc_layer(N, R, acc_rows, BN, False)
    h = tc(outB1, outB1, outB1, outB1, degPad, x,
           W1[:, :64, :], W1[:, 64:, :], Wroot1,
           jnp.zeros((8, 128), F32).at[0].set(b1))

    hr = h.reshape(2 * N, 64)
    sc2 = _build_sc_segsum(N, R, E_pad, acc_rows, False, nbuf=5)
    (outB2,) = sc2(hr, combR)

    tc2 = _build_tc_layer(N, R, acc_rows, BN, True)
    out = tc2(outB2, outB2, outB2, outB2, degPad, h,
              W2[:, :64, :], W2[:, 64:, :], Wroot2,
              jnp.zeros((8, 128), F32).at[0].set(b2))
    return out
